# baseline (speedup 1.0000x reference)
import functools

import jax
import jax.numpy as jnp
from jax import lax
from jax.experimental import pallas as pl
from jax.experimental.pallas import tpu as pltpu
from jax.experimental.pallas import tpu_sc as plsc

NUM_HIDDENS = 128
MAX_LEN = 32768
N_IDS = 16384

_NC = 2
_NS = 16
_NW = _NC * _NS
_B_PER_W = N_IDS // _NW
_CHUNK = 32
_NCHUNKS = _B_PER_W // _CHUNK

_mesh = plsc.VectorSubcoreMesh(core_axis_name="c", subcore_axis_name="s")


@functools.partial(
    pl.kernel,
    mesh=_mesh,
    out_type=jax.ShapeDtypeStruct((N_IDS, NUM_HIDDENS), jnp.float32),
    scratch_types=[
        pltpu.VMEM((_B_PER_W,), jnp.int32),
        pltpu.VMEM((_B_PER_W, NUM_HIDDENS), jnp.float32),
        pltpu.SemaphoreType.DMA,
    ],
)
def _gather_rows(table_hbm, idx_hbm, out_hbm, idx_v, rows_v, sem):
    wid = lax.axis_index("s") * _NC + lax.axis_index("c")
    base = wid * _B_PER_W
    pltpu.sync_copy(idx_hbm.at[pl.ds(base, _B_PER_W)], idx_v)
    copies = [
        pltpu.async_copy(
            table_hbm.at[idx_v.at[pl.ds(j * _CHUNK, _CHUNK)]],
            rows_v.at[pl.ds(j * _CHUNK, _CHUNK)],
            sem,
        )
        for j in range(_NCHUNKS)
    ]
    for c in copies:
        c.wait()
    pltpu.sync_copy(rows_v, out_hbm.at[pl.ds(base, _B_PER_W)])


def kernel(position_ids, P):
    table = P.reshape(MAX_LEN, NUM_HIDDENS)
    out = _gather_rows(table, position_ids)
    return out.reshape(1, N_IDS, NUM_HIDDENS)


# speedup vs baseline: 1.0292x; 1.0292x over previous
import functools

import jax
import jax.numpy as jnp
from jax import lax
from jax.experimental import pallas as pl
from jax.experimental.pallas import tpu as pltpu
from jax.experimental.pallas import tpu_sc as plsc

NUM_HIDDENS = 128
MAX_LEN = 32768
N_IDS = 16384

_NC = 2
_NS = 16
_NW = _NC * _NS
_B_PER_W = N_IDS // _NW
_CHUNK = 64
_NCHUNKS = _B_PER_W // _CHUNK

_mesh = plsc.VectorSubcoreMesh(core_axis_name="c", subcore_axis_name="s")


@functools.partial(
    pl.kernel,
    mesh=_mesh,
    out_type=jax.ShapeDtypeStruct((N_IDS, NUM_HIDDENS), jnp.float32),
    scratch_types=[
        pltpu.VMEM((_B_PER_W,), jnp.int32),
        pltpu.VMEM((_B_PER_W, NUM_HIDDENS), jnp.float32),
        pltpu.SemaphoreType.DMA,
    ],
)
def _gather_rows(table_hbm, idx_hbm, out_hbm, idx_v, rows_v, sem):
    wid = lax.axis_index("s") * _NC + lax.axis_index("c")
    base = wid * _B_PER_W
    pltpu.sync_copy(idx_hbm.at[pl.ds(base, _B_PER_W)], idx_v)
    copies = [
        pltpu.async_copy(
            table_hbm.at[idx_v.at[pl.ds(j * _CHUNK, _CHUNK)]],
            rows_v.at[pl.ds(j * _CHUNK, _CHUNK)],
            sem,
        )
        for j in range(_NCHUNKS)
    ]
    for c in copies:
        c.wait()
    pltpu.sync_copy(rows_v, out_hbm.at[pl.ds(base, _B_PER_W)])


def kernel(position_ids, P):
    table = P.reshape(MAX_LEN, NUM_HIDDENS)
    out = _gather_rows(table, position_ids)
    return out.reshape(1, N_IDS, NUM_HIDDENS)
